# baseline (speedup 1.0000x reference)
import functools

import jax
import jax.numpy as jnp
from jax import lax
from jax.experimental import pallas as pl
from jax.experimental.pallas import tpu as pltpu
from jax.experimental.pallas import tpu_sc as plsc

ALPHA = 0.1
KHOPS = 5
NC = 2
NS = 16
NW = NC * NS
CH = 128
DEGW = 16


def _cdiv(a, b):
    return (a + b - 1) // b


def _row_block(n):
    for r in (1000, 500, 250, 200, 125, 100, 80, 50, 40, 25, 20, 16, 10, 8, 5, 4, 2, 1):
        if n % r == 0:
            return r
    return 1


def _mlp(x, W1, b1, W2, b2):
    n, d_in = x.shape
    d_hid = W1.shape[1]
    d_out = W2.shape[1]
    rows = _row_block(n)

    def body(x_r, w1_r, b1_r, w2_r, b2_r, h_r):
        a = jnp.dot(x_r[...], w1_r[...], preferred_element_type=jnp.float32)
        a = jnp.maximum(a + b1_r[...], 0.0)
        h_r[...] = jnp.dot(a, w2_r[...], preferred_element_type=jnp.float32) + b2_r[...]

    return pl.pallas_call(
        body,
        grid=(n // rows,),
        in_specs=[
            pl.BlockSpec((rows, d_in), lambda i: (i, 0)),
            pl.BlockSpec((d_in, d_hid), lambda i: (0, 0)),
            pl.BlockSpec((1, d_hid), lambda i: (0, 0)),
            pl.BlockSpec((d_hid, d_out), lambda i: (0, 0)),
            pl.BlockSpec((1, d_out), lambda i: (0, 0)),
        ],
        out_specs=pl.BlockSpec((rows, d_out), lambda i: (i, 0)),
        out_shape=jax.ShapeDtypeStruct((n, d_out), jnp.float32),
    )(x, W1.astype(jnp.float32), b1.reshape(1, -1), W2.astype(jnp.float32), b2.reshape(1, -1))


def _prep(degw, h):
    n, d = h.shape
    rows = _row_block(n)

    def body(dw0_r, dw1_r, h_r, y_r, hd_r, d2_r, sq_r):
        deg = dw0_r[0][:, 0:1] + dw1_r[0][:, 0:1] + 1.0
        dinv = lax.rsqrt(deg)
        hb = h_r[...]
        y_r[...] = dinv * hb
        hd_r[...] = ALPHA * (dinv * hb)
        ones = jnp.ones_like(hb)
        d2_r[...] = ((1.0 - ALPHA) * (dinv * dinv)) * ones
        sq_r[...] = jnp.sqrt(deg) * ones

    o = jax.ShapeDtypeStruct((n, d), jnp.float32)
    return pl.pallas_call(
        body,
        grid=(n // rows,),
        in_specs=[
            pl.BlockSpec((1, rows, DEGW), lambda i: (0, i, 0)),
            pl.BlockSpec((1, rows, DEGW), lambda i: (1, i, 0)),
            pl.BlockSpec((rows, d), lambda i: (i, 0)),
        ],
        out_specs=[pl.BlockSpec((rows, d), lambda i: (i, 0))] * 4,
        out_shape=[o, o, o, o],
    )(degw, degw, h)


def _update(raw, y, d2, hd):
    n, d = y.shape
    rows = _row_block(n)

    def body(r0_r, r1_r, y_r, d2_r, hd_r, o_r):
        o_r[...] = d2_r[...] * (r0_r[0] + r1_r[0] + y_r[...]) + hd_r[...]

    return pl.pallas_call(
        body,
        grid=(n // rows,),
        in_specs=[
            pl.BlockSpec((1, rows, d), lambda i: (0, i, 0)),
            pl.BlockSpec((1, rows, d), lambda i: (1, i, 0)),
            pl.BlockSpec((rows, d), lambda i: (i, 0)),
            pl.BlockSpec((rows, d), lambda i: (i, 0)),
            pl.BlockSpec((rows, d), lambda i: (i, 0)),
        ],
        out_specs=pl.BlockSpec((rows, d), lambda i: (i, 0)),
        out_shape=jax.ShapeDtypeStruct((n, d), jnp.float32),
    )(raw, raw, y, d2, hd)


def _final(raw, y, d2, hd, sq):
    n, d = y.shape
    rows = _row_block(n)

    def body(r0_r, r1_r, y_r, d2_r, hd_r, sq_r, o_r):
        ynext = d2_r[...] * (r0_r[0] + r1_r[0] + y_r[...]) + hd_r[...]
        z = ynext * sq_r[...]
        m = jnp.max(z, axis=1, keepdims=True)
        zs = z - m
        o_r[...] = zs - jnp.log(jnp.sum(jnp.exp(zs), axis=1, keepdims=True))

    return pl.pallas_call(
        body,
        grid=(n // rows,),
        in_specs=[
            pl.BlockSpec((1, rows, d), lambda i: (0, i, 0)),
            pl.BlockSpec((1, rows, d), lambda i: (1, i, 0)),
            pl.BlockSpec((rows, d), lambda i: (i, 0)),
            pl.BlockSpec((rows, d), lambda i: (i, 0)),
            pl.BlockSpec((rows, d), lambda i: (i, 0)),
            pl.BlockSpec((rows, d), lambda i: (i, 0)),
        ],
        out_specs=pl.BlockSpec((rows, d), lambda i: (i, 0)),
        out_shape=jax.ShapeDtypeStruct((n, d), jnp.float32),
    )(raw, raw, y, d2, hd, sq)


def _deg_call(dst3, npad, cpt):
    rpt = npad // NS
    mesh = plsc.VectorSubcoreMesh(
        core_axis_name="c", subcore_axis_name="s", num_cores=NC, num_subcores=NS
    )

    @functools.partial(
        pl.kernel,
        out_type=jax.ShapeDtypeStruct((NC, npad, DEGW), jnp.float32),
        mesh=mesh,
        compiler_params=pltpu.CompilerParams(use_tc_tiling_on_sc=False),
        scratch_types=[
            pltpu.VMEM((cpt, CH), jnp.int32),
            pltpu.VMEM((CH, DEGW), jnp.float32),
            pltpu.VMEM((rpt, DEGW), jnp.float32),
            pltpu.VMEM_SHARED((npad, DEGW), jnp.float32),
        ],
    )
    def body(dst_hbm, degw_hbm, dst_v, ones_v, zbuf, acc):
        c = lax.axis_index("c")
        s = lax.axis_index("s")
        w = c * NS + s
        zv = jnp.zeros((16,), jnp.float32)
        ov = jnp.ones((16,), jnp.float32)

        def zrow(i, carry):
            zbuf[i, pl.ds(0, DEGW)] = zv[pl.ds(0, DEGW)] if DEGW != 16 else zv
            return carry

        lax.fori_loop(0, rpt, zrow, 0)

        def orow(i, carry):
            ones_v[i, pl.ds(0, DEGW)] = ov
            return carry

        lax.fori_loop(0, CH, orow, 0)
        pltpu.sync_copy(zbuf, acc.at[pl.ds(s * rpt, rpt)])
        pltpu.sync_copy(dst_hbm.at[w], dst_v)
        plsc.subcore_barrier()

        def chunk(i, carry):
            pltpu.sync_copy(ones_v, acc.at[dst_v.at[i]], add=True)
            return carry

        lax.fori_loop(0, cpt, chunk, 0)
        plsc.subcore_barrier()
        pltpu.sync_copy(acc.at[pl.ds(s * rpt, rpt)], zbuf)
        pltpu.sync_copy(zbuf, degw_hbm.at[c, pl.ds(s * rpt, rpt)])

    return body(dst3)


def _edge_call(y, src3, dst3, npad, cpt, d, nb=2):
    rpt = npad // NS
    mesh = plsc.VectorSubcoreMesh(
        core_axis_name="c", subcore_axis_name="s", num_cores=NC, num_subcores=NS
    )

    @functools.partial(
        pl.kernel,
        out_type=jax.ShapeDtypeStruct((NC, npad, d), jnp.float32),
        mesh=mesh,
        compiler_params=pltpu.CompilerParams(use_tc_tiling_on_sc=False),
        scratch_types=[
            pltpu.VMEM((cpt, CH), jnp.int32),
            pltpu.VMEM((cpt, CH), jnp.int32),
            [pltpu.VMEM((CH, d), jnp.float32) for _ in range(nb)],
            pltpu.VMEM((rpt, d), jnp.float32),
            pltpu.VMEM_SHARED((npad, d), jnp.float32),
            [pltpu.SemaphoreType.DMA for _ in range(nb)],
        ],
    )
    def body(y_hbm, src_hbm, dst_hbm, raw_hbm,
             src_v, dst_v, bufs, zbuf, acc, gsem):
        c = lax.axis_index("c")
        s = lax.axis_index("s")
        w = c * NS + s
        zv = jnp.zeros((16,), jnp.float32)

        def zrow(i, carry):
            for jj in range(d // 16):
                zbuf[i, pl.ds(jj * 16, 16)] = zv
            return carry

        lax.fori_loop(0, rpt, zrow, 0)
        pltpu.sync_copy(zbuf, acc.at[pl.ds(s * rpt, rpt)])
        pltpu.sync_copy(src_hbm.at[w], src_v)
        pltpu.sync_copy(dst_hbm.at[w], dst_v)
        plsc.subcore_barrier()
        for b in range(nb):
            pltpu.async_copy(y_hbm.at[src_v.at[b]], bufs[b], gsem[b])

        def quad(j, carry):
            i0 = nb * j
            for b in range(nb):
                pltpu.make_async_copy(
                    y_hbm.at[src_v.at[i0 + b]], bufs[b], gsem[b]).wait()
                pltpu.sync_copy(bufs[b], acc.at[dst_v.at[i0 + b]], add=True)

                @pl.when(i0 + nb + b < cpt)
                def _():
                    pltpu.async_copy(
                        y_hbm.at[src_v.at[i0 + nb + b]], bufs[b], gsem[b])

            return carry

        lax.fori_loop(0, cpt // nb, quad, 0)
        plsc.subcore_barrier()
        pltpu.sync_copy(acc.at[pl.ds(s * rpt, rpt)], zbuf)
        pltpu.sync_copy(zbuf, raw_hbm.at[c, pl.ds(s * rpt, rpt)])

    return body(y, src3, dst3)


def kernel(x, edge_index, W1, b1, W2, b2):
    n = x.shape[0]
    d = W2.shape[1]
    e = edge_index.shape[1]

    npad = _cdiv(n + 1, NS * 8) * NS * 8
    rpt = npad // NS
    nb = 3
    cpt = _cdiv(e, NW * CH)
    cpt = _cdiv(cpt, nb) * nb
    tot = NW * cpt * CH
    pad = tot - e

    src3 = jnp.concatenate(
        [edge_index[0], jnp.zeros((pad,), jnp.int32)]).reshape(NW, cpt, CH)
    dst3 = jnp.concatenate(
        [edge_index[1], jnp.full((pad,), n, jnp.int32)]).reshape(NW, cpt, CH)
    h = _mlp(x, W1, b1, W2, b2)
    degw = _deg_call(dst3, npad, cpt)
    y, hd, d2, sq = _prep(degw, h)
    for k in range(KHOPS):
        raw = _edge_call(y, src3, dst3, npad, cpt, d, nb)
        if k < KHOPS - 1:
            y = _update(raw, y, d2, hd)
        else:
            out = _final(raw, y, d2, hd, sq)
    return out


# speedup vs baseline: 1.3613x; 1.3613x over previous
import functools

import jax
import jax.numpy as jnp
from jax import lax
from jax.experimental import pallas as pl
from jax.experimental.pallas import tpu as pltpu
from jax.experimental.pallas import tpu_sc as plsc

ALPHA = 0.1
KHOPS = 5
NC = 2
NS = 16
NW = NC * NS
CH = 128
DEGW = 16


def _cdiv(a, b):
    return (a + b - 1) // b


def _row_block(n):
    for r in (1000, 500, 250, 200, 125, 100, 80, 50, 40, 25, 20, 16, 10, 8, 5, 4, 2, 1):
        if n % r == 0:
            return r
    return 1


def _mlp(x, W1, b1, W2, b2):
    n, d_in = x.shape
    d_hid = W1.shape[1]
    d_out = W2.shape[1]
    rows = _row_block(n)

    def body(x_r, w1_r, b1_r, w2_r, b2_r, h_r):
        a = jnp.dot(x_r[...], w1_r[...], preferred_element_type=jnp.float32)
        a = jnp.maximum(a + b1_r[...], 0.0)
        h_r[...] = jnp.dot(a, w2_r[...], preferred_element_type=jnp.float32) + b2_r[...]

    return pl.pallas_call(
        body,
        grid=(n // rows,),
        in_specs=[
            pl.BlockSpec((rows, d_in), lambda i: (i, 0)),
            pl.BlockSpec((d_in, d_hid), lambda i: (0, 0)),
            pl.BlockSpec((1, d_hid), lambda i: (0, 0)),
            pl.BlockSpec((d_hid, d_out), lambda i: (0, 0)),
            pl.BlockSpec((1, d_out), lambda i: (0, 0)),
        ],
        out_specs=pl.BlockSpec((rows, d_out), lambda i: (i, 0)),
        out_shape=jax.ShapeDtypeStruct((n, d_out), jnp.float32),
    )(x, W1.astype(jnp.float32), b1.reshape(1, -1), W2.astype(jnp.float32), b2.reshape(1, -1))


def _prep(degw, h):
    n, d = h.shape
    rows = _row_block(n)

    def body(dw0_r, dw1_r, h_r, y_r, hd_r, d2_r, sq_r):
        deg = dw0_r[0][:, 0:1] + dw1_r[0][:, 0:1] + 1.0
        dinv = lax.rsqrt(deg)
        hb = h_r[...]
        y_r[...] = dinv * hb
        hd_r[...] = ALPHA * (dinv * hb)
        ones = jnp.ones_like(hb)
        d2_r[...] = ((1.0 - ALPHA) * (dinv * dinv)) * ones
        sq_r[...] = jnp.sqrt(deg) * ones

    o = jax.ShapeDtypeStruct((n, d), jnp.float32)
    return pl.pallas_call(
        body,
        grid=(n // rows,),
        in_specs=[
            pl.BlockSpec((1, rows, DEGW), lambda i: (0, i, 0)),
            pl.BlockSpec((1, rows, DEGW), lambda i: (1, i, 0)),
            pl.BlockSpec((rows, d), lambda i: (i, 0)),
        ],
        out_specs=[pl.BlockSpec((rows, d), lambda i: (i, 0))] * 4,
        out_shape=[o, o, o, o],
    )(degw, degw, h)


def _update(raw, y, d2, hd):
    n, d = y.shape
    rows = _row_block(n)

    def body(r0_r, r1_r, y_r, d2_r, hd_r, o_r):
        o_r[...] = d2_r[...] * (r0_r[0] + r1_r[0] + y_r[...]) + hd_r[...]

    return pl.pallas_call(
        body,
        grid=(n // rows,),
        in_specs=[
            pl.BlockSpec((1, rows, d), lambda i: (0, i, 0)),
            pl.BlockSpec((1, rows, d), lambda i: (1, i, 0)),
            pl.BlockSpec((rows, d), lambda i: (i, 0)),
            pl.BlockSpec((rows, d), lambda i: (i, 0)),
            pl.BlockSpec((rows, d), lambda i: (i, 0)),
        ],
        out_specs=pl.BlockSpec((rows, d), lambda i: (i, 0)),
        out_shape=jax.ShapeDtypeStruct((n, d), jnp.float32),
    )(raw, raw, y, d2, hd)


def _final(raw, y, d2, hd, sq):
    n, d = y.shape
    rows = _row_block(n)

    def body(r0_r, r1_r, y_r, d2_r, hd_r, sq_r, o_r):
        ynext = d2_r[...] * (r0_r[0] + r1_r[0] + y_r[...]) + hd_r[...]
        z = ynext * sq_r[...]
        m = jnp.max(z, axis=1, keepdims=True)
        zs = z - m
        o_r[...] = zs - jnp.log(jnp.sum(jnp.exp(zs), axis=1, keepdims=True))

    return pl.pallas_call(
        body,
        grid=(n // rows,),
        in_specs=[
            pl.BlockSpec((1, rows, d), lambda i: (0, i, 0)),
            pl.BlockSpec((1, rows, d), lambda i: (1, i, 0)),
            pl.BlockSpec((rows, d), lambda i: (i, 0)),
            pl.BlockSpec((rows, d), lambda i: (i, 0)),
            pl.BlockSpec((rows, d), lambda i: (i, 0)),
            pl.BlockSpec((rows, d), lambda i: (i, 0)),
        ],
        out_specs=pl.BlockSpec((rows, d), lambda i: (i, 0)),
        out_shape=jax.ShapeDtypeStruct((n, d), jnp.float32),
    )(raw, raw, y, d2, hd, sq)


def _deg_call(dst3, npad, cpt):
    rpt = npad // NS
    mesh = plsc.VectorSubcoreMesh(
        core_axis_name="c", subcore_axis_name="s", num_cores=NC, num_subcores=NS
    )

    @functools.partial(
        pl.kernel,
        out_type=jax.ShapeDtypeStruct((NC, npad, DEGW), jnp.float32),
        mesh=mesh,
        compiler_params=pltpu.CompilerParams(use_tc_tiling_on_sc=False),
        scratch_types=[
            pltpu.VMEM((cpt, CH), jnp.int32),
            pltpu.VMEM((CH, DEGW), jnp.float32),
            pltpu.VMEM((rpt, DEGW), jnp.float32),
            pltpu.VMEM_SHARED((npad, DEGW), jnp.float32),
        ],
    )
    def body(dst_hbm, degw_hbm, dst_v, ones_v, zbuf, acc):
        c = lax.axis_index("c")
        s = lax.axis_index("s")
        w = c * NS + s
        zv = jnp.zeros((16,), jnp.float32)
        ov = jnp.ones((16,), jnp.float32)

        def zrow(i, carry):
            zbuf[i, pl.ds(0, DEGW)] = zv[pl.ds(0, DEGW)] if DEGW != 16 else zv
            return carry

        lax.fori_loop(0, rpt, zrow, 0)

        def orow(i, carry):
            ones_v[i, pl.ds(0, DEGW)] = ov
            return carry

        lax.fori_loop(0, CH, orow, 0)
        pltpu.sync_copy(zbuf, acc.at[pl.ds(s * rpt, rpt)])
        pltpu.sync_copy(dst_hbm.at[w], dst_v)
        plsc.subcore_barrier()

        def chunk(i, carry):
            pltpu.sync_copy(ones_v, acc.at[dst_v.at[i]], add=True)
            return carry

        lax.fori_loop(0, cpt, chunk, 0)
        plsc.subcore_barrier()
        pltpu.sync_copy(acc.at[pl.ds(s * rpt, rpt)], zbuf)
        pltpu.sync_copy(zbuf, degw_hbm.at[c, pl.ds(s * rpt, rpt)])

    return body(dst3)


def _edge_call(y, src3, dst3, npad, cpt, d, nb=2):
    rpt = npad // NS
    mesh = plsc.VectorSubcoreMesh(
        core_axis_name="c", subcore_axis_name="s", num_cores=NC, num_subcores=NS
    )

    @functools.partial(
        pl.kernel,
        out_type=jax.ShapeDtypeStruct((NC, npad, d), jnp.float32),
        mesh=mesh,
        compiler_params=pltpu.CompilerParams(use_tc_tiling_on_sc=False),
        scratch_types=[
            pltpu.VMEM((cpt, CH), jnp.int32),
            pltpu.VMEM((cpt, CH), jnp.int32),
            [pltpu.VMEM((CH, d), jnp.float32) for _ in range(nb)],
            pltpu.VMEM((rpt, d), jnp.float32),
            pltpu.VMEM_SHARED((npad, d), jnp.float32),
            [pltpu.SemaphoreType.DMA for _ in range(nb)],
        ],
    )
    def body(y_hbm, src_hbm, dst_hbm, raw_hbm,
             src_v, dst_v, bufs, zbuf, acc, gsem):
        c = lax.axis_index("c")
        s = lax.axis_index("s")
        w = c * NS + s
        zv = jnp.zeros((16,), jnp.float32)

        def zrow(i, carry):
            for jj in range(d // 16):
                zbuf[i, pl.ds(jj * 16, 16)] = zv
            return carry

        lax.fori_loop(0, rpt, zrow, 0)
        pltpu.sync_copy(zbuf, acc.at[pl.ds(s * rpt, rpt)])
        pltpu.sync_copy(src_hbm.at[w], src_v)
        pltpu.sync_copy(dst_hbm.at[w], dst_v)
        plsc.subcore_barrier()
        for b in range(nb):
            pltpu.async_copy(y_hbm.at[src_v.at[b]], bufs[b], gsem[b])

        def quad(j, carry):
            i0 = nb * j
            for b in range(nb):
                pltpu.make_async_copy(
                    y_hbm.at[src_v.at[i0 + b]], bufs[b], gsem[b]).wait()
                pltpu.sync_copy(bufs[b], acc.at[dst_v.at[i0 + b]], add=True)

                @pl.when(i0 + nb + b < cpt)
                def _():
                    pltpu.async_copy(
                        y_hbm.at[src_v.at[i0 + nb + b]], bufs[b], gsem[b])

            return carry

        lax.fori_loop(0, cpt // nb, quad, 0)
        plsc.subcore_barrier()
        pltpu.sync_copy(acc.at[pl.ds(s * rpt, rpt)], zbuf)
        pltpu.sync_copy(zbuf, raw_hbm.at[c, pl.ds(s * rpt, rpt)])

    return body(y, src3, dst3)


def kernel(x, edge_index, W1, b1, W2, b2):
    n = x.shape[0]
    d = W2.shape[1]
    e = edge_index.shape[1]

    npad = _cdiv(n + 1, NS * 8) * NS * 8
    rpt = npad // NS
    nb = 2
    cpt = _cdiv(e, NW * CH)
    cpt = _cdiv(cpt, nb) * nb
    tot = NW * cpt * CH
    pad = tot - e

    src3 = jnp.concatenate(
        [edge_index[0], jnp.zeros((pad,), jnp.int32)]).reshape(NW, cpt, CH)
    dst3 = jnp.concatenate(
        [edge_index[1], jnp.full((pad,), n, jnp.int32)]).reshape(NW, cpt, CH)
    h = _mlp(x, W1, b1, W2, b2)
    degw = _deg_call(dst3, npad, cpt)
    y, hd, d2, sq = _prep(degw, h)
    for k in range(KHOPS):
        raw = _edge_call(y, src3, dst3, npad, cpt, d, nb)
        if k < KHOPS - 1:
            y = _update(raw, y, d2, hd)
        else:
            out = _final(raw, y, d2, hd, sq)
    return out


# speedup vs baseline: 2.8922x; 2.1245x over previous
import functools

import jax
import jax.numpy as jnp
from jax import lax
from jax.experimental import pallas as pl
from jax.experimental.pallas import tpu as pltpu
from jax.experimental.pallas import tpu_sc as plsc

ALPHA = 0.1
KHOPS = 5
NC = 2
NS = 16
NW = NC * NS
CH = 128
DEGW = 16


def _cdiv(a, b):
    return (a + b - 1) // b


def _row_block(n):
    for r in (1000, 500, 250, 200, 125, 100, 80, 50, 40, 25, 20, 16, 10, 8, 5, 4, 2, 1):
        if n % r == 0:
            return r
    return 1


def _mlp(x, W1, b1, W2, b2):
    n, d_in = x.shape
    d_hid = W1.shape[1]
    d_out = W2.shape[1]
    rows = _row_block(n)

    def body(x_r, w1_r, b1_r, w2_r, b2_r, h_r):
        a = jnp.dot(x_r[...], w1_r[...], preferred_element_type=jnp.float32)
        a = jnp.maximum(a + b1_r[...], 0.0)
        h_r[...] = jnp.dot(a, w2_r[...], preferred_element_type=jnp.float32) + b2_r[...]

    return pl.pallas_call(
        body,
        grid=(n // rows,),
        in_specs=[
            pl.BlockSpec((rows, d_in), lambda i: (i, 0)),
            pl.BlockSpec((d_in, d_hid), lambda i: (0, 0)),
            pl.BlockSpec((1, d_hid), lambda i: (0, 0)),
            pl.BlockSpec((d_hid, d_out), lambda i: (0, 0)),
            pl.BlockSpec((1, d_out), lambda i: (0, 0)),
        ],
        out_specs=pl.BlockSpec((rows, d_out), lambda i: (i, 0)),
        out_shape=jax.ShapeDtypeStruct((n, d_out), jnp.float32),
    )(x, W1.astype(jnp.float32), b1.reshape(1, -1), W2.astype(jnp.float32), b2.reshape(1, -1))


def _prep(degw, h):
    n, d = h.shape
    rows = n // NS if n % (NS * 8) == 0 else _row_block(n)

    def body(dw0_r, dw1_r, h_r, y_r, hd_r, d2_r, sq_r):
        deg = dw0_r[0][:, 0:1] + dw1_r[0][:, 0:1] + 1.0
        dinv = lax.rsqrt(deg)
        hb = h_r[...]
        y_r[...] = dinv * hb
        hd_r[...] = ALPHA * (dinv * hb)
        ones = jnp.ones_like(hb)
        d2_r[...] = ((1.0 - ALPHA) * (dinv * dinv)) * ones
        sq_r[...] = jnp.sqrt(deg) * ones

    o = jax.ShapeDtypeStruct((n, d), jnp.float32)
    return pl.pallas_call(
        body,
        grid=(n // rows,),
        in_specs=[
            pl.BlockSpec((1, rows, DEGW), lambda i: (0, i, 0)),
            pl.BlockSpec((1, rows, DEGW), lambda i: (1, i, 0)),
            pl.BlockSpec((rows, d), lambda i: (i, 0)),
        ],
        out_specs=[pl.BlockSpec((rows, d), lambda i: (i, 0))] * 4,
        out_shape=[o, o, o, o],
    )(degw, degw, h)


def _update(raw, y, d2, hd):
    n, d = y.shape
    rows = n // NS if n % (NS * 8) == 0 else _row_block(n)

    def body(r0_r, r1_r, y_r, d2_r, hd_r, o_r):
        o_r[...] = d2_r[...] * (r0_r[0] + r1_r[0] + y_r[...]) + hd_r[...]

    return pl.pallas_call(
        body,
        grid=(n // rows,),
        in_specs=[
            pl.BlockSpec((1, rows, d), lambda i: (0, i, 0)),
            pl.BlockSpec((1, rows, d), lambda i: (1, i, 0)),
            pl.BlockSpec((rows, d), lambda i: (i, 0)),
            pl.BlockSpec((rows, d), lambda i: (i, 0)),
            pl.BlockSpec((rows, d), lambda i: (i, 0)),
        ],
        out_specs=pl.BlockSpec((rows, d), lambda i: (i, 0)),
        out_shape=jax.ShapeDtypeStruct((n, d), jnp.float32),
    )(raw, raw, y, d2, hd)


def _final(raw, y, d2, hd, sq):
    n, d = y.shape
    rows = n // NS if n % (NS * 8) == 0 else _row_block(n)

    def body(r0_r, r1_r, y_r, d2_r, hd_r, sq_r, o_r):
        ynext = d2_r[...] * (r0_r[0] + r1_r[0] + y_r[...]) + hd_r[...]
        z = ynext * sq_r[...]
        m = jnp.max(z, axis=1, keepdims=True)
        zs = z - m
        o_r[...] = zs - jnp.log(jnp.sum(jnp.exp(zs), axis=1, keepdims=True))

    return pl.pallas_call(
        body,
        grid=(n // rows,),
        in_specs=[
            pl.BlockSpec((1, rows, d), lambda i: (0, i, 0)),
            pl.BlockSpec((1, rows, d), lambda i: (1, i, 0)),
            pl.BlockSpec((rows, d), lambda i: (i, 0)),
            pl.BlockSpec((rows, d), lambda i: (i, 0)),
            pl.BlockSpec((rows, d), lambda i: (i, 0)),
            pl.BlockSpec((rows, d), lambda i: (i, 0)),
        ],
        out_specs=pl.BlockSpec((rows, d), lambda i: (i, 0)),
        out_shape=jax.ShapeDtypeStruct((n, d), jnp.float32),
    )(raw, raw, y, d2, hd, sq)


def _deg_call(dst3, npad, cpt):
    rpt = npad // NS
    mesh = plsc.VectorSubcoreMesh(
        core_axis_name="c", subcore_axis_name="s", num_cores=NC, num_subcores=NS
    )

    @functools.partial(
        pl.kernel,
        out_type=jax.ShapeDtypeStruct((NC, npad, DEGW), jnp.float32),
        mesh=mesh,
        compiler_params=pltpu.CompilerParams(use_tc_tiling_on_sc=False),
        scratch_types=[
            pltpu.VMEM((cpt, CH), jnp.int32),
            pltpu.VMEM((CH, DEGW), jnp.float32),
            pltpu.VMEM((rpt, DEGW), jnp.float32),
            pltpu.VMEM_SHARED((npad, DEGW), jnp.float32),
        ],
    )
    def body(dst_hbm, degw_hbm, dst_v, ones_v, zbuf, acc):
        c = lax.axis_index("c")
        s = lax.axis_index("s")
        w = c * NS + s
        zv = jnp.zeros((16,), jnp.float32)
        ov = jnp.ones((16,), jnp.float32)

        def zrow(i, carry):
            zbuf[i, pl.ds(0, DEGW)] = zv[pl.ds(0, DEGW)] if DEGW != 16 else zv
            return carry

        lax.fori_loop(0, rpt, zrow, 0)

        def orow(i, carry):
            ones_v[i, pl.ds(0, DEGW)] = ov
            return carry

        lax.fori_loop(0, CH, orow, 0)
        pltpu.sync_copy(zbuf, acc.at[pl.ds(s * rpt, rpt)])
        pltpu.sync_copy(dst_hbm.at[w], dst_v)
        plsc.subcore_barrier()

        def chunk(i, carry):
            pltpu.sync_copy(ones_v, acc.at[dst_v.at[i]], add=True)
            return carry

        lax.fori_loop(0, cpt, chunk, 0)
        plsc.subcore_barrier()
        pltpu.sync_copy(acc.at[pl.ds(s * rpt, rpt)], zbuf)
        pltpu.sync_copy(zbuf, degw_hbm.at[c, pl.ds(s * rpt, rpt)])

    return body(dst3)


def _edge_call(y, src3, dst3, npad, cpt, d, nb=2):
    rpt = npad // NS
    base = (rpt // 4) // 8 * 8
    chunks = []
    off = 0
    while off < rpt:
        sz = min(base, rpt - off)
        chunks.append((off, sz))
        off += sz
    mesh = plsc.VectorSubcoreMesh(
        core_axis_name="c", subcore_axis_name="s", num_cores=NC, num_subcores=NS
    )

    @functools.partial(
        pl.kernel,
        out_type=jax.ShapeDtypeStruct((NC, npad, d), jnp.float32),
        mesh=mesh,
        compiler_params=pltpu.CompilerParams(use_tc_tiling_on_sc=False),
        scratch_types=[
            pltpu.VMEM((cpt, CH), jnp.int32),
            pltpu.VMEM((cpt, CH), jnp.int32),
            [pltpu.VMEM((CH, d), jnp.float32) for _ in range(nb)],
            pltpu.VMEM((base, d), jnp.float32),
            pltpu.VMEM_SHARED((npad, d), jnp.float32),
            pltpu.VMEM_SHARED((npad, d), jnp.float32),
            [pltpu.SemaphoreType.DMA for _ in range(nb)],
        ],
    )
    def body(y_hbm, src_hbm, dst_hbm, raw_hbm,
             src_v, dst_v, bufs, stg, acc, ysp, gsem):
        c = lax.axis_index("c")
        s = lax.axis_index("s")
        w = c * NS + s
        r0 = s * rpt
        for (o, sz) in chunks:
            pltpu.sync_copy(y_hbm.at[pl.ds(r0 + o, sz)], stg.at[pl.ds(0, sz)])
            pltpu.sync_copy(stg.at[pl.ds(0, sz)], ysp.at[pl.ds(r0 + o, sz)])
        zv = jnp.zeros((16,), jnp.float32)

        def zrow(i, carry):
            for jj in range(d // 16):
                stg[i, pl.ds(jj * 16, 16)] = zv
            return carry

        lax.fori_loop(0, base, zrow, 0)
        for (o, sz) in chunks:
            pltpu.sync_copy(stg.at[pl.ds(0, sz)], acc.at[pl.ds(r0 + o, sz)])
        pltpu.sync_copy(src_hbm.at[w], src_v)
        pltpu.sync_copy(dst_hbm.at[w], dst_v)
        plsc.subcore_barrier()
        for b in range(nb):
            pltpu.async_copy(ysp.at[src_v.at[b]], bufs[b], gsem[b])

        def quad(j, carry):
            i0 = nb * j
            for b in range(nb):
                pltpu.make_async_copy(
                    ysp.at[src_v.at[i0 + b]], bufs[b], gsem[b]).wait()
                pltpu.sync_copy(bufs[b], acc.at[dst_v.at[i0 + b]], add=True)

                @pl.when(i0 + nb + b < cpt)
                def _():
                    pltpu.async_copy(
                        ysp.at[src_v.at[i0 + nb + b]], bufs[b], gsem[b])

            return carry

        lax.fori_loop(0, cpt // nb, quad, 0)
        plsc.subcore_barrier()
        for (o, sz) in chunks:
            pltpu.sync_copy(acc.at[pl.ds(r0 + o, sz)], stg.at[pl.ds(0, sz)])
            pltpu.sync_copy(stg.at[pl.ds(0, sz)], raw_hbm.at[c, pl.ds(r0 + o, sz)])

    return body(y, src3, dst3)


def kernel(x, edge_index, W1, b1, W2, b2):
    n = x.shape[0]
    d = W2.shape[1]
    e = edge_index.shape[1]

    npad = _cdiv(n + 1, NS * 8) * NS * 8
    rpt = npad // NS
    nb = 2
    cpt = _cdiv(e, NW * CH)
    cpt = _cdiv(cpt, nb) * nb
    tot = NW * cpt * CH
    pad = tot - e

    src3 = jnp.concatenate(
        [edge_index[0], jnp.zeros((pad,), jnp.int32)]).reshape(NW, cpt, CH)
    dst3 = jnp.concatenate(
        [edge_index[1], jnp.full((pad,), n, jnp.int32)]).reshape(NW, cpt, CH)
    h = _mlp(x, W1, b1, W2, b2)
    hp = jnp.pad(h, ((0, npad - n), (0, 0)))
    degw = _deg_call(dst3, npad, cpt)
    y, hd, d2, sq = _prep(degw, hp)
    for k in range(KHOPS):
        raw = _edge_call(y, src3, dst3, npad, cpt, d, nb)
        if k < KHOPS - 1:
            y = _update(raw, y, d2, hd)
        else:
            out = _final(raw, y, d2, hd, sq)
    return out[:n]


# speedup vs baseline: 3.1331x; 1.0833x over previous
import functools

import jax
import jax.numpy as jnp
from jax import lax
from jax.experimental import pallas as pl
from jax.experimental.pallas import tpu as pltpu
from jax.experimental.pallas import tpu_sc as plsc

ALPHA = 0.1
KHOPS = 5
NC = 2
NS = 16
NW = NC * NS
CH = 64
DEGW = 16


def _cdiv(a, b):
    return (a + b - 1) // b


def _row_block(n):
    for r in (1000, 500, 250, 200, 125, 100, 80, 50, 40, 25, 20, 16, 10, 8, 5, 4, 2, 1):
        if n % r == 0:
            return r
    return 1


def _mlp(x, W1, b1, W2, b2):
    n, d_in = x.shape
    d_hid = W1.shape[1]
    d_out = W2.shape[1]
    rows = _row_block(n)

    def body(x_r, w1_r, b1_r, w2_r, b2_r, h_r):
        a = jnp.dot(x_r[...], w1_r[...], preferred_element_type=jnp.float32)
        a = jnp.maximum(a + b1_r[...], 0.0)
        h_r[...] = jnp.dot(a, w2_r[...], preferred_element_type=jnp.float32) + b2_r[...]

    return pl.pallas_call(
        body,
        grid=(n // rows,),
        in_specs=[
            pl.BlockSpec((rows, d_in), lambda i: (i, 0)),
            pl.BlockSpec((d_in, d_hid), lambda i: (0, 0)),
            pl.BlockSpec((1, d_hid), lambda i: (0, 0)),
            pl.BlockSpec((d_hid, d_out), lambda i: (0, 0)),
            pl.BlockSpec((1, d_out), lambda i: (0, 0)),
        ],
        out_specs=pl.BlockSpec((rows, d_out), lambda i: (i, 0)),
        out_shape=jax.ShapeDtypeStruct((n, d_out), jnp.float32),
    )(x, W1.astype(jnp.float32), b1.reshape(1, -1), W2.astype(jnp.float32), b2.reshape(1, -1))


def _prep(degw, h):
    n, d = h.shape
    rows = n // NS if n % (NS * 8) == 0 else _row_block(n)

    def body(dw0_r, dw1_r, h_r, y_r, hd_r, d2_r, sq_r):
        deg = dw0_r[0][:, 0:1] + dw1_r[0][:, 0:1] + 1.0
        dinv = lax.rsqrt(deg)
        hb = h_r[...]
        y_r[...] = dinv * hb
        hd_r[...] = ALPHA * (dinv * hb)
        ones = jnp.ones_like(hb)
        d2_r[...] = ((1.0 - ALPHA) * (dinv * dinv)) * ones
        sq_r[...] = jnp.sqrt(deg) * ones

    o = jax.ShapeDtypeStruct((n, d), jnp.float32)
    return pl.pallas_call(
        body,
        grid=(n // rows,),
        in_specs=[
            pl.BlockSpec((1, rows, DEGW), lambda i: (0, i, 0)),
            pl.BlockSpec((1, rows, DEGW), lambda i: (1, i, 0)),
            pl.BlockSpec((rows, d), lambda i: (i, 0)),
        ],
        out_specs=[pl.BlockSpec((rows, d), lambda i: (i, 0))] * 4,
        out_shape=[o, o, o, o],
    )(degw, degw, h)


def _update(raw, y, d2, hd):
    n, d = y.shape
    rows = n // NS if n % (NS * 8) == 0 else _row_block(n)

    def body(r0_r, r1_r, y_r, d2_r, hd_r, o_r):
        o_r[...] = d2_r[...] * (r0_r[0] + r1_r[0] + y_r[...]) + hd_r[...]

    return pl.pallas_call(
        body,
        grid=(n // rows,),
        in_specs=[
            pl.BlockSpec((1, rows, d), lambda i: (0, i, 0)),
            pl.BlockSpec((1, rows, d), lambda i: (1, i, 0)),
            pl.BlockSpec((rows, d), lambda i: (i, 0)),
            pl.BlockSpec((rows, d), lambda i: (i, 0)),
            pl.BlockSpec((rows, d), lambda i: (i, 0)),
        ],
        out_specs=pl.BlockSpec((rows, d), lambda i: (i, 0)),
        out_shape=jax.ShapeDtypeStruct((n, d), jnp.float32),
    )(raw, raw, y, d2, hd)


def _final(raw, y, d2, hd, sq):
    n, d = y.shape
    rows = n // NS if n % (NS * 8) == 0 else _row_block(n)

    def body(r0_r, r1_r, y_r, d2_r, hd_r, sq_r, o_r):
        ynext = d2_r[...] * (r0_r[0] + r1_r[0] + y_r[...]) + hd_r[...]
        z = ynext * sq_r[...]
        m = jnp.max(z, axis=1, keepdims=True)
        zs = z - m
        o_r[...] = zs - jnp.log(jnp.sum(jnp.exp(zs), axis=1, keepdims=True))

    return pl.pallas_call(
        body,
        grid=(n // rows,),
        in_specs=[
            pl.BlockSpec((1, rows, d), lambda i: (0, i, 0)),
            pl.BlockSpec((1, rows, d), lambda i: (1, i, 0)),
            pl.BlockSpec((rows, d), lambda i: (i, 0)),
            pl.BlockSpec((rows, d), lambda i: (i, 0)),
            pl.BlockSpec((rows, d), lambda i: (i, 0)),
            pl.BlockSpec((rows, d), lambda i: (i, 0)),
        ],
        out_specs=pl.BlockSpec((rows, d), lambda i: (i, 0)),
        out_shape=jax.ShapeDtypeStruct((n, d), jnp.float32),
    )(raw, raw, y, d2, hd, sq)


def _deg_call(dst3, npad, cpt):
    rpt = npad // NS
    mesh = plsc.VectorSubcoreMesh(
        core_axis_name="c", subcore_axis_name="s", num_cores=NC, num_subcores=NS
    )

    @functools.partial(
        pl.kernel,
        out_type=jax.ShapeDtypeStruct((NC, npad, DEGW), jnp.float32),
        mesh=mesh,
        compiler_params=pltpu.CompilerParams(use_tc_tiling_on_sc=False),
        scratch_types=[
            pltpu.VMEM((cpt, CH), jnp.int32),
            pltpu.VMEM((CH, DEGW), jnp.float32),
            pltpu.VMEM((rpt, DEGW), jnp.float32),
            pltpu.VMEM_SHARED((npad, DEGW), jnp.float32),
        ],
    )
    def body(dst_hbm, degw_hbm, dst_v, ones_v, zbuf, acc):
        c = lax.axis_index("c")
        s = lax.axis_index("s")
        w = c * NS + s
        zv = jnp.zeros((16,), jnp.float32)
        ov = jnp.ones((16,), jnp.float32)

        def zrow(i, carry):
            zbuf[i, pl.ds(0, DEGW)] = zv[pl.ds(0, DEGW)] if DEGW != 16 else zv
            return carry

        lax.fori_loop(0, rpt, zrow, 0)

        def orow(i, carry):
            ones_v[i, pl.ds(0, DEGW)] = ov
            return carry

        lax.fori_loop(0, CH, orow, 0)
        pltpu.sync_copy(zbuf, acc.at[pl.ds(s * rpt, rpt)])
        pltpu.sync_copy(dst_hbm.at[w], dst_v)
        plsc.subcore_barrier()

        def chunk(i, carry):
            pltpu.sync_copy(ones_v, acc.at[dst_v.at[i]], add=True)
            return carry

        lax.fori_loop(0, cpt, chunk, 0)
        plsc.subcore_barrier()
        pltpu.sync_copy(acc.at[pl.ds(s * rpt, rpt)], zbuf)
        pltpu.sync_copy(zbuf, degw_hbm.at[c, pl.ds(s * rpt, rpt)])

    return body(dst3)


def _edge_call(y, src3, dst3, npad, cpt, d, nb=2):
    rpt = npad // NS
    base = min(128, rpt // 8 * 8)
    chunks = []
    off = 0
    while off < rpt:
        sz = min(base, rpt - off)
        chunks.append((off, sz))
        off += sz
    mesh = plsc.VectorSubcoreMesh(
        core_axis_name="c", subcore_axis_name="s", num_cores=NC, num_subcores=NS
    )

    @functools.partial(
        pl.kernel,
        out_type=jax.ShapeDtypeStruct((NC, npad, d), jnp.float32),
        mesh=mesh,
        compiler_params=pltpu.CompilerParams(use_tc_tiling_on_sc=False),
        scratch_types=[
            pltpu.VMEM((cpt, CH), jnp.int32),
            pltpu.VMEM((cpt, CH), jnp.int32),
            [pltpu.VMEM((CH, d), jnp.float32) for _ in range(nb)],
            pltpu.VMEM((base, d), jnp.float32),
            pltpu.VMEM_SHARED((npad, d), jnp.float32),
            pltpu.VMEM_SHARED((npad, d), jnp.float32),
            [pltpu.SemaphoreType.DMA for _ in range(nb)],
            [pltpu.SemaphoreType.DMA for _ in range(nb)],
        ],
    )
    def body(y_hbm, src_hbm, dst_hbm, raw_hbm,
             src_v, dst_v, bufs, stg, acc, ysp, gsem, ssem):
        c = lax.axis_index("c")
        s = lax.axis_index("s")
        w = c * NS + s
        r0 = s * rpt
        for (o, sz) in chunks:
            pltpu.sync_copy(y_hbm.at[pl.ds(r0 + o, sz)], stg.at[pl.ds(0, sz)])
            pltpu.sync_copy(stg.at[pl.ds(0, sz)], ysp.at[pl.ds(r0 + o, sz)])
        zv = jnp.zeros((16,), jnp.float32)

        def zrow(i, carry):
            for jj in range(d // 16):
                stg[i, pl.ds(jj * 16, 16)] = zv
            return carry

        lax.fori_loop(0, base, zrow, 0)
        for (o, sz) in chunks:
            pltpu.sync_copy(stg.at[pl.ds(0, sz)], acc.at[pl.ds(r0 + o, sz)])
        pltpu.sync_copy(src_hbm.at[w], src_v)
        pltpu.sync_copy(dst_hbm.at[w], dst_v)
        plsc.subcore_barrier()
        pltpu.async_copy(ysp.at[src_v.at[0]], bufs[0], gsem[0])
        pltpu.async_copy(ysp.at[src_v.at[1]], bufs[1], gsem[1])

        def ring(j, carry):
            i0 = nb * j
            for k in range(nb):
                i = i0 + k
                kw = (k + 2) % nb
                pltpu.make_async_copy(
                    ysp.at[src_v.at[i]], bufs[k], gsem[k]).wait()
                pltpu.async_copy(
                    bufs[k], acc.at[dst_v.at[i]], ssem[k], add=True)

                @pl.when(i >= 2)
                def _():
                    pltpu.make_async_copy(
                        bufs[kw], acc.at[dst_v.at[i]], ssem[kw]).wait()

                @pl.when(i + 2 < cpt)
                def _():
                    pltpu.async_copy(
                        ysp.at[src_v.at[i + 2]], bufs[kw], gsem[kw])

            return carry

        lax.fori_loop(0, cpt // nb, ring, 0)
        for k in ((cpt - 2) % nb, (cpt - 1) % nb):
            pltpu.make_async_copy(bufs[k], acc.at[dst_v.at[0]], ssem[k]).wait()
        plsc.subcore_barrier()
        for (o, sz) in chunks:
            pltpu.sync_copy(acc.at[pl.ds(r0 + o, sz)], stg.at[pl.ds(0, sz)])
            pltpu.sync_copy(stg.at[pl.ds(0, sz)], raw_hbm.at[c, pl.ds(r0 + o, sz)])

    return body(y, src3, dst3)


def kernel(x, edge_index, W1, b1, W2, b2):
    n = x.shape[0]
    d = W2.shape[1]
    e = edge_index.shape[1]

    npad = _cdiv(n + 1, NS * 8) * NS * 8
    rpt = npad // NS
    nb = 4
    cpt = _cdiv(e, NW * CH)
    cpt = _cdiv(cpt, nb) * nb
    tot = NW * cpt * CH
    pad = tot - e

    src3 = jnp.concatenate(
        [edge_index[0], jnp.zeros((pad,), jnp.int32)]).reshape(NW, cpt, CH)
    dst3 = jnp.concatenate(
        [edge_index[1], jnp.full((pad,), n, jnp.int32)]).reshape(NW, cpt, CH)
    h = _mlp(x, W1, b1, W2, b2)
    hp = jnp.pad(h, ((0, npad - n), (0, 0)))
    degw = _deg_call(dst3, npad, cpt)
    y, hd, d2, sq = _prep(degw, hp)
    for k in range(KHOPS):
        raw = _edge_call(y, src3, dst3, npad, cpt, d, nb)
        if k < KHOPS - 1:
            y = _update(raw, y, d2, hd)
        else:
            out = _final(raw, y, d2, hd, sq)
    return out[:n]


# speedup vs baseline: 3.2703x; 1.0438x over previous
import functools

import jax
import jax.numpy as jnp
from jax import lax
from jax.experimental import pallas as pl
from jax.experimental.pallas import tpu as pltpu
from jax.experimental.pallas import tpu_sc as plsc

ALPHA = 0.1
KHOPS = 5
NC = 2
NS = 16
NW = NC * NS
CH = 128
DEGW = 16


def _cdiv(a, b):
    return (a + b - 1) // b


def _row_block(n):
    for r in (1000, 500, 250, 200, 125, 100, 80, 50, 40, 25, 20, 16, 10, 8, 5, 4, 2, 1):
        if n % r == 0:
            return r
    return 1


def _mlp(x, W1, b1, W2, b2):
    n, d_in = x.shape
    d_hid = W1.shape[1]
    d_out = W2.shape[1]
    rows = _row_block(n)

    def body(x_r, w1_r, b1_r, w2_r, b2_r, h_r):
        a = jnp.dot(x_r[...], w1_r[...], preferred_element_type=jnp.float32)
        a = jnp.maximum(a + b1_r[...], 0.0)
        h_r[...] = jnp.dot(a, w2_r[...], preferred_element_type=jnp.float32) + b2_r[...]

    return pl.pallas_call(
        body,
        grid=(n // rows,),
        in_specs=[
            pl.BlockSpec((rows, d_in), lambda i: (i, 0)),
            pl.BlockSpec((d_in, d_hid), lambda i: (0, 0)),
            pl.BlockSpec((1, d_hid), lambda i: (0, 0)),
            pl.BlockSpec((d_hid, d_out), lambda i: (0, 0)),
            pl.BlockSpec((1, d_out), lambda i: (0, 0)),
        ],
        out_specs=pl.BlockSpec((rows, d_out), lambda i: (i, 0)),
        out_shape=jax.ShapeDtypeStruct((n, d_out), jnp.float32),
    )(x, W1.astype(jnp.float32), b1.reshape(1, -1), W2.astype(jnp.float32), b2.reshape(1, -1))


def _prep(degw, h):
    n, d = h.shape
    rows = n // 4 if n % 32 == 0 else _row_block(n)

    def body(dw0_r, dw1_r, h_r, y_r, hd_r, d2_r, sq_r):
        deg = dw0_r[0][:, 0:1] + dw1_r[0][:, 0:1] + 1.0
        dinv = lax.rsqrt(deg)
        hb = h_r[...]
        y_r[...] = dinv * hb
        hd_r[...] = ALPHA * (dinv * hb)
        ones = jnp.ones_like(hb)
        d2_r[...] = ((1.0 - ALPHA) * (dinv * dinv)) * ones
        sq_r[...] = jnp.sqrt(deg) * ones

    o = jax.ShapeDtypeStruct((n, d), jnp.float32)
    return pl.pallas_call(
        body,
        grid=(n // rows,),
        in_specs=[
            pl.BlockSpec((1, rows, DEGW), lambda i: (0, i, 0)),
            pl.BlockSpec((1, rows, DEGW), lambda i: (1, i, 0)),
            pl.BlockSpec((rows, d), lambda i: (i, 0)),
        ],
        out_specs=[pl.BlockSpec((rows, d), lambda i: (i, 0))] * 4,
        out_shape=[o, o, o, o],
    )(degw, degw, h)


def _update(raw, y, d2, hd):
    n, d = y.shape
    rows = n // 4 if n % 32 == 0 else _row_block(n)

    def body(r0_r, r1_r, y_r, d2_r, hd_r, o_r):
        o_r[...] = d2_r[...] * (r0_r[0] + r1_r[0] + y_r[...]) + hd_r[...]

    return pl.pallas_call(
        body,
        grid=(n // rows,),
        in_specs=[
            pl.BlockSpec((1, rows, d), lambda i: (0, i, 0)),
            pl.BlockSpec((1, rows, d), lambda i: (1, i, 0)),
            pl.BlockSpec((rows, d), lambda i: (i, 0)),
            pl.BlockSpec((rows, d), lambda i: (i, 0)),
            pl.BlockSpec((rows, d), lambda i: (i, 0)),
        ],
        out_specs=pl.BlockSpec((rows, d), lambda i: (i, 0)),
        out_shape=jax.ShapeDtypeStruct((n, d), jnp.float32),
    )(raw, raw, y, d2, hd)


def _final(raw, y, d2, hd, sq):
    n, d = y.shape
    rows = n // 4 if n % 32 == 0 else _row_block(n)

    def body(r0_r, r1_r, y_r, d2_r, hd_r, sq_r, o_r):
        ynext = d2_r[...] * (r0_r[0] + r1_r[0] + y_r[...]) + hd_r[...]
        z = ynext * sq_r[...]
        m = jnp.max(z, axis=1, keepdims=True)
        zs = z - m
        o_r[...] = zs - jnp.log(jnp.sum(jnp.exp(zs), axis=1, keepdims=True))

    return pl.pallas_call(
        body,
        grid=(n // rows,),
        in_specs=[
            pl.BlockSpec((1, rows, d), lambda i: (0, i, 0)),
            pl.BlockSpec((1, rows, d), lambda i: (1, i, 0)),
            pl.BlockSpec((rows, d), lambda i: (i, 0)),
            pl.BlockSpec((rows, d), lambda i: (i, 0)),
            pl.BlockSpec((rows, d), lambda i: (i, 0)),
            pl.BlockSpec((rows, d), lambda i: (i, 0)),
        ],
        out_specs=pl.BlockSpec((rows, d), lambda i: (i, 0)),
        out_shape=jax.ShapeDtypeStruct((n, d), jnp.float32),
    )(raw, raw, y, d2, hd, sq)


def _deg_call(dst3, npad, cpt):
    rpt = npad // NS
    mesh = plsc.VectorSubcoreMesh(
        core_axis_name="c", subcore_axis_name="s", num_cores=NC, num_subcores=NS
    )

    @functools.partial(
        pl.kernel,
        out_type=jax.ShapeDtypeStruct((NC, npad, DEGW), jnp.float32),
        mesh=mesh,
        compiler_params=pltpu.CompilerParams(use_tc_tiling_on_sc=False),
        scratch_types=[
            pltpu.VMEM((cpt, CH), jnp.int32),
            pltpu.VMEM((CH, DEGW), jnp.float32),
            pltpu.VMEM((rpt, DEGW), jnp.float32),
            pltpu.VMEM_SHARED((npad, DEGW), jnp.float32),
        ],
    )
    def body(dst_hbm, degw_hbm, dst_v, ones_v, zbuf, acc):
        c = lax.axis_index("c")
        s = lax.axis_index("s")
        w = c * NS + s
        zv = jnp.zeros((16,), jnp.float32)
        ov = jnp.ones((16,), jnp.float32)

        def zrow(i, carry):
            zbuf[i, pl.ds(0, DEGW)] = zv[pl.ds(0, DEGW)] if DEGW != 16 else zv
            return carry

        lax.fori_loop(0, rpt, zrow, 0)

        def orow(i, carry):
            ones_v[i, pl.ds(0, DEGW)] = ov
            return carry

        lax.fori_loop(0, CH, orow, 0)
        pltpu.sync_copy(zbuf, acc.at[pl.ds(s * rpt, rpt)])
        pltpu.sync_copy(dst_hbm.at[w], dst_v)
        plsc.subcore_barrier()

        def chunk(i, carry):
            pltpu.sync_copy(ones_v, acc.at[dst_v.at[i]], add=True)
            return carry

        lax.fori_loop(0, cpt, chunk, 0)
        plsc.subcore_barrier()
        pltpu.sync_copy(acc.at[pl.ds(s * rpt, rpt)], zbuf)
        pltpu.sync_copy(zbuf, degw_hbm.at[c, pl.ds(s * rpt, rpt)])

    return body(dst3)


def _edge_call(y, src3, dst3, npad, cpt, d, nb=2):
    rpt = npad // NS
    base = CH
    chunks = []
    off = 0
    while off < rpt:
        sz = min(base, rpt - off)
        chunks.append((off, sz))
        off += sz
    mesh = plsc.VectorSubcoreMesh(
        core_axis_name="c", subcore_axis_name="s", num_cores=NC, num_subcores=NS
    )

    @functools.partial(
        pl.kernel,
        out_type=jax.ShapeDtypeStruct((NC, npad, d), jnp.float32),
        mesh=mesh,
        compiler_params=pltpu.CompilerParams(use_tc_tiling_on_sc=False),
        scratch_types=[
            pltpu.VMEM((cpt, CH), jnp.int32),
            pltpu.VMEM((cpt, CH), jnp.int32),
            [pltpu.VMEM((CH, d), jnp.float32) for _ in range(nb)],
            pltpu.VMEM_SHARED((npad, d), jnp.float32),
            pltpu.VMEM_SHARED((npad, d), jnp.float32),
            [pltpu.SemaphoreType.DMA for _ in range(nb)],
            [pltpu.SemaphoreType.DMA for _ in range(nb)],
        ],
    )
    def body(y_hbm, src_hbm, dst_hbm, raw_hbm,
             src_v, dst_v, bufs, acc, ysp, gsem, ssem):
        c = lax.axis_index("c")
        s = lax.axis_index("s")
        w = c * NS + s
        r0 = s * rpt
        for ci, (o, sz) in enumerate(chunks):
            b = bufs[ci % 2]
            pltpu.sync_copy(y_hbm.at[pl.ds(r0 + o, sz)], b.at[pl.ds(0, sz)])
            pltpu.sync_copy(b.at[pl.ds(0, sz)], ysp.at[pl.ds(r0 + o, sz)])
        zv = jnp.zeros((16,), jnp.float32)
        zb = bufs[2]

        def zrow(i, carry):
            for jj in range(d // 16):
                zb[i, pl.ds(jj * 16, 16)] = zv
            return carry

        lax.fori_loop(0, base, zrow, 0)
        for (o, sz) in chunks:
            pltpu.sync_copy(zb.at[pl.ds(0, sz)], acc.at[pl.ds(r0 + o, sz)])
        pltpu.sync_copy(src_hbm.at[w], src_v)
        pltpu.sync_copy(dst_hbm.at[w], dst_v)
        plsc.subcore_barrier()
        pltpu.async_copy(ysp.at[src_v.at[0]], bufs[0], gsem[0])
        pltpu.async_copy(ysp.at[src_v.at[1]], bufs[1], gsem[1])

        def ring(j, carry):
            i0 = nb * j
            for k in range(nb):
                i = i0 + k
                kw = (k + 2) % nb
                pltpu.make_async_copy(
                    ysp.at[src_v.at[i]], bufs[k], gsem[k]).wait()
                pltpu.async_copy(
                    bufs[k], acc.at[dst_v.at[i]], ssem[k], add=True)

                @pl.when(i + 2 >= nb)
                def _():
                    pltpu.make_async_copy(
                        bufs[kw], acc.at[dst_v.at[i]], ssem[kw]).wait()

                @pl.when(i + 2 < cpt)
                def _():
                    pltpu.async_copy(
                        ysp.at[src_v.at[i + 2]], bufs[kw], gsem[kw])

            return carry

        lax.fori_loop(0, cpt // nb, ring, 0)
        for t in range(cpt - (nb - 2), cpt):
            pltpu.make_async_copy(
                bufs[t % nb], acc.at[dst_v.at[0]], ssem[t % nb]).wait()
        plsc.subcore_barrier()
        for ci, (o, sz) in enumerate(chunks):
            b = bufs[ci % 2]
            pltpu.sync_copy(acc.at[pl.ds(r0 + o, sz)], b.at[pl.ds(0, sz)])
            pltpu.sync_copy(b.at[pl.ds(0, sz)], raw_hbm.at[c, pl.ds(r0 + o, sz)])

    return body(y, src3, dst3)


def kernel(x, edge_index, W1, b1, W2, b2):
    n = x.shape[0]
    d = W2.shape[1]
    e = edge_index.shape[1]

    npad = _cdiv(n + 1, NS * 8) * NS * 8
    rpt = npad // NS
    nb = 3
    cpt = _cdiv(e, NW * CH)
    cpt = _cdiv(cpt, nb) * nb
    tot = NW * cpt * CH
    pad = tot - e

    src3 = jnp.concatenate(
        [edge_index[0], jnp.zeros((pad,), jnp.int32)]).reshape(NW, cpt, CH)
    dst3 = jnp.concatenate(
        [edge_index[1], jnp.full((pad,), n, jnp.int32)]).reshape(NW, cpt, CH)
    h = _mlp(x, W1, b1, W2, b2)
    hp = jnp.pad(h, ((0, npad - n), (0, 0)))
    degw = _deg_call(dst3, npad, cpt)
    y, hd, d2, sq = _prep(degw, hp)
    for k in range(KHOPS):
        raw = _edge_call(y, src3, dst3, npad, cpt, d, nb)
        if k < KHOPS - 1:
            y = _update(raw, y, d2, hd)
        else:
            out = _final(raw, y, d2, hd, sq)
    return out[:n]
